# baseline (speedup 1.0000x reference)
import functools

import jax
import jax.numpy as jnp
from jax import lax
from jax.experimental import pallas as pl
from jax.experimental.pallas import tpu as pltpu

POOL = 1000000
D = 64
QN = 64
K = 8
BLK = 8192
NB = 123
SEGW = 64
NSEG = POOL // SEGW
SEGB = BLK // SEGW
NSEGP = NB * SEGB
NCAND = K * SEGW
NEG = -1e30
BIG = 2**30


def _k1_body(qt_ref, kb_ref, out_ref):
    b = pl.program_id(0)
    kb = kb_ref[...]
    scores = lax.dot_general(kb, qt_ref[...], (((1,), (0,)), ((), ())),
                             preferred_element_type=jnp.float32)
    smax = jnp.max(scores.reshape(SEGB, SEGW, QN), axis=1)
    out_ref[...] = smax

    @pl.when(b == NB - 1)
    def _():
        seg = lax.broadcasted_iota(jnp.int32, (SEGB, QN), 0)
        out_ref[...] = jnp.where(b * SEGB + seg >= NSEG, NEG, smax)


def _k2_body(seg_ref, ids_ref):
    x = seg_ref[...]
    iota = lax.broadcasted_iota(jnp.int32, (NSEGP, QN), 0)
    ids = []
    for _ in range(K):
        m = jnp.max(x, axis=0, keepdims=True)
        sel = x == m
        idx = jnp.min(jnp.where(sel, iota, BIG), axis=0, keepdims=True)
        ids.append(idx)
        x = jnp.where(iota == idx, NEG, x)
    ids_ref[...] = jnp.concatenate(ids, axis=0)


def _k4_body(seg_ref, q_ref, k0, k1, k2, k3, k4, k5, k6, k7,
             idx_ref, w_ref):
    r = pl.program_id(0)
    qr = q_ref[0]
    kcat = jnp.concatenate([k[...] for k in (k0, k1, k2, k3, k4, k5, k6, k7)],
                           axis=0)
    sc = lax.dot_general(qr, kcat, (((1,), (1,)), ((), ())),
                         preferred_element_type=jnp.float32)
    j = lax.broadcasted_iota(jnp.int32, (1, SEGW), 1)
    ci = jnp.concatenate(
        [seg_ref[r, i] * SEGW + j for i in range(K)], axis=1)
    vals, idxs = [], []
    for _ in range(K):
        m = jnp.max(sc, axis=1, keepdims=True)
        sel = sc == m
        pidx = jnp.min(jnp.where(sel, ci, BIG), axis=1, keepdims=True)
        vals.append(m)
        idxs.append(pidx)
        sc = jnp.where(ci == pidx, NEG, sc)
    v = jnp.concatenate(vals, axis=1)
    idx_ref[...] = jnp.concatenate(idxs, axis=1).reshape(1, 1, K)
    e = jnp.exp(v - v[:, 0:1])
    w_ref[...] = (e / jnp.sum(e, axis=1, keepdims=True)).reshape(1, 1, K)


def _k5_body(t_ref, u_ref, wb_ref, p0, p1, p2, p3, p4, p5, p6, p7,
             out_ref):
    r = pl.program_id(0)
    tiles = (p0, p1, p2, p3, p4, p5, p6, p7)
    rows = []
    for i in range(K):
        u = u_ref[r, i]
        rows.append(tiles[i][0, pl.ds(u, 1), :])
    stack = jnp.concatenate(rows, axis=0)
    agg = jnp.sum(stack * wb_ref[0], axis=0)
    out_ref[...] = agg.reshape(1, 1, D)


def _k6_body(agg_ref, W_ref, out_ref):
    out_ref[...] = lax.dot_general(agg_ref[...], W_ref[...],
                                   (((1,), (1,)), ((), ())),
                                   preferred_element_type=jnp.float32)


@jax.jit
def kernel(query, pool, keys, W):
    B, S, _ = query.shape
    q = query.reshape(QN, D)
    qt = q.T

    seg_max = pl.pallas_call(
        _k1_body,
        grid=(NB,),
        in_specs=[
            pl.BlockSpec((D, QN), lambda b: (0, 0)),
            pl.BlockSpec((BLK, D), lambda b: (b, 0)),
        ],
        out_specs=pl.BlockSpec((SEGB, QN), lambda b: (b, 0)),
        out_shape=jax.ShapeDtypeStruct((NSEGP, QN), jnp.float32),
    )(qt, keys)

    seg_ids_t = pl.pallas_call(
        _k2_body,
        out_shape=jax.ShapeDtypeStruct((K, QN), jnp.int32),
    )(seg_max)

    seg = seg_ids_t.T
    q3 = q.reshape(QN, 1, D)

    def _seg_map(i):
        return lambda r, seg_ref: (seg_ref[r, i], 0)

    final_idx3, weights3 = pl.pallas_call(
        _k4_body,
        grid_spec=pltpu.PrefetchScalarGridSpec(
            num_scalar_prefetch=1,
            grid=(QN,),
            in_specs=[pl.BlockSpec((1, 1, D), lambda r, s: (r, 0, 0))] +
                     [pl.BlockSpec((SEGW, D), _seg_map(i)) for i in range(K)],
            out_specs=[pl.BlockSpec((1, 1, K), lambda r, s: (r, 0, 0)),
                       pl.BlockSpec((1, 1, K), lambda r, s: (r, 0, 0))],
        ),
        out_shape=[jax.ShapeDtypeStruct((QN, 1, K), jnp.int32),
                   jax.ShapeDtypeStruct((QN, 1, K), jnp.float32)],
    )(seg, q3, *([keys] * K))

    final_idx = final_idx3.reshape(QN, K)
    tidx = final_idx // 8
    uoff = final_idx % 8
    wb = jnp.broadcast_to(weights3.reshape(QN, K, 1), (QN, K, D))
    pool3 = pool.reshape(POOL // 8, 8, D)

    def _tile_map(i):
        return lambda r, t_ref, u_ref: (t_ref[r, i], 0, 0)

    agg3 = pl.pallas_call(
        _k5_body,
        grid_spec=pltpu.PrefetchScalarGridSpec(
            num_scalar_prefetch=2,
            grid=(QN,),
            in_specs=[pl.BlockSpec((1, K, D), lambda r, t, u: (r, 0, 0))] +
                     [pl.BlockSpec((1, 8, D), _tile_map(i)) for i in range(K)],
            out_specs=pl.BlockSpec((1, 1, D), lambda r, t, u: (r, 0, 0)),
        ),
        out_shape=jax.ShapeDtypeStruct((QN, 1, D), jnp.float32),
    )(tidx, uoff, wb, *([pool3] * K))

    out = pl.pallas_call(
        _k6_body,
        out_shape=jax.ShapeDtypeStruct((QN, D), jnp.float32),
    )(agg3.reshape(QN, D), W)
    return out.reshape(B, S, D)


# speedup vs baseline: 1.1494x; 1.1494x over previous
import functools

import jax
import jax.numpy as jnp
from jax import lax
from jax.experimental import pallas as pl
from jax.experimental.pallas import tpu as pltpu

POOL = 1000000
D = 64
QN = 64
K = 8
BLK = 8192
NB = 123
SEGW = 64
NSEG = POOL // SEGW
SEGB = BLK // SEGW
NSEGP = NB * SEGB
NCAND = K * SEGW
NEG = -1e30
BIG = 2**30


def _k1_body(qt_ref, kb_ref, out_ref):
    b = pl.program_id(0)
    kb = kb_ref[...].reshape(BLK, D)
    scores = lax.dot_general(kb, qt_ref[...], (((1,), (0,)), ((), ())),
                             preferred_element_type=jnp.float32)
    smax = jnp.max(scores.reshape(SEGB, SEGW, QN), axis=1)
    out_ref[...] = smax

    @pl.when(b == NB - 1)
    def _():
        seg = lax.broadcasted_iota(jnp.int32, (SEGB, QN), 0)
        out_ref[...] = jnp.where(b * SEGB + seg >= NSEG, NEG, smax)


def _k2_body(seg_ref, ids_ref):
    x = seg_ref[...]
    iota = lax.broadcasted_iota(jnp.int32, (NSEGP, QN), 0)
    ids = []
    for _ in range(K):
        m = jnp.max(x, axis=0, keepdims=True)
        sel = x == m
        idx = jnp.min(jnp.where(sel, iota, BIG), axis=0, keepdims=True)
        ids.append(idx)
        x = jnp.where(iota == idx, NEG, x)
    ids_ref[...] = jnp.concatenate(ids, axis=0)


def _k4_body(seg_ref, q_ref, k0, k1, k2, k3, k4, k5, k6, k7,
             idx_ref, w_ref):
    r = pl.program_id(0)
    qr = q_ref[0]
    kcat = jnp.concatenate(
        [k[...].reshape(SEGW, D) for k in (k0, k1, k2, k3, k4, k5, k6, k7)],
        axis=0)
    sc = lax.dot_general(qr, kcat, (((1,), (1,)), ((), ())),
                         preferred_element_type=jnp.float32)
    j = lax.broadcasted_iota(jnp.int32, (1, SEGW), 1)
    ci = jnp.concatenate(
        [seg_ref[r, i] * SEGW + j for i in range(K)], axis=1)
    vals, idxs = [], []
    for _ in range(K):
        m = jnp.max(sc, axis=1, keepdims=True)
        sel = sc == m
        pidx = jnp.min(jnp.where(sel, ci, BIG), axis=1, keepdims=True)
        vals.append(m)
        idxs.append(pidx)
        sc = jnp.where(ci == pidx, NEG, sc)
    v = jnp.concatenate(vals, axis=1)
    idx_ref[...] = jnp.concatenate(idxs, axis=1).reshape(1, 1, K)
    e = jnp.exp(v - v[:, 0:1])
    w_ref[...] = (e / jnp.sum(e, axis=1, keepdims=True)).reshape(1, 1, K)


def _k5_body(t_ref, u_ref, wb_ref, p0, p1, p2, p3, p4, p5, p6, p7,
             out_ref):
    r = pl.program_id(0)
    tiles = (p0, p1, p2, p3, p4, p5, p6, p7)
    rows = []
    for i in range(K):
        u = u_ref[r, i]
        rows.append(tiles[i][0, pl.ds(u, 1), :])
    stack = jnp.concatenate(rows, axis=0)
    agg = jnp.sum(stack * wb_ref[0], axis=0)
    out_ref[...] = agg.reshape(1, 1, D)


def _k6_body(agg_ref, W_ref, out_ref):
    out_ref[...] = lax.dot_general(agg_ref[...], W_ref[...],
                                   (((1,), (1,)), ((), ())),
                                   preferred_element_type=jnp.float32)


@jax.jit
def kernel(query, pool, keys, W):
    B, S, _ = query.shape
    q = query.reshape(QN, D)
    qt = q.T
    keys3 = keys.reshape(POOL // 8, 8, D)

    seg_max = pl.pallas_call(
        _k1_body,
        grid=(NB,),
        in_specs=[
            pl.BlockSpec((D, QN), lambda b: (0, 0)),
            pl.BlockSpec((BLK // 8, 8, D), lambda b: (b, 0, 0)),
        ],
        out_specs=pl.BlockSpec((SEGB, QN), lambda b: (b, 0)),
        out_shape=jax.ShapeDtypeStruct((NSEGP, QN), jnp.float32),
    )(qt, keys3)

    seg_ids_t = pl.pallas_call(
        _k2_body,
        out_shape=jax.ShapeDtypeStruct((K, QN), jnp.int32),
    )(seg_max)

    seg = seg_ids_t.T
    q3 = q.reshape(QN, 1, D)

    def _seg_map(i):
        return lambda r, seg_ref: (seg_ref[r, i], 0, 0)

    final_idx3, weights3 = pl.pallas_call(
        _k4_body,
        grid_spec=pltpu.PrefetchScalarGridSpec(
            num_scalar_prefetch=1,
            grid=(QN,),
            in_specs=[pl.BlockSpec((1, 1, D), lambda r, s: (r, 0, 0))] +
                     [pl.BlockSpec((SEGW // 8, 8, D), _seg_map(i))
                      for i in range(K)],
            out_specs=[pl.BlockSpec((1, 1, K), lambda r, s: (r, 0, 0)),
                       pl.BlockSpec((1, 1, K), lambda r, s: (r, 0, 0))],
        ),
        out_shape=[jax.ShapeDtypeStruct((QN, 1, K), jnp.int32),
                   jax.ShapeDtypeStruct((QN, 1, K), jnp.float32)],
    )(seg, q3, *([keys3] * K))

    final_idx = final_idx3.reshape(QN, K)
    tidx = final_idx // 8
    uoff = final_idx % 8
    wb = jnp.broadcast_to(weights3.reshape(QN, K, 1), (QN, K, D))
    pool3 = pool.reshape(POOL // 8, 8, D)

    def _tile_map(i):
        return lambda r, t_ref, u_ref: (t_ref[r, i], 0, 0)

    agg3 = pl.pallas_call(
        _k5_body,
        grid_spec=pltpu.PrefetchScalarGridSpec(
            num_scalar_prefetch=2,
            grid=(QN,),
            in_specs=[pl.BlockSpec((1, K, D), lambda r, t, u: (r, 0, 0))] +
                     [pl.BlockSpec((1, 8, D), _tile_map(i)) for i in range(K)],
            out_specs=pl.BlockSpec((1, 1, D), lambda r, t, u: (r, 0, 0)),
        ),
        out_shape=jax.ShapeDtypeStruct((QN, 1, D), jnp.float32),
    )(tidx, uoff, wb, *([pool3] * K))

    out = pl.pallas_call(
        _k6_body,
        out_shape=jax.ShapeDtypeStruct((QN, D), jnp.float32),
    )(agg3.reshape(QN, D), W)
    return out.reshape(B, S, D)


# speedup vs baseline: 2.2338x; 1.9435x over previous
import functools

import jax
import jax.numpy as jnp
from jax import lax
from jax.experimental import pallas as pl
from jax.experimental.pallas import tpu as pltpu

POOL = 1000000
D = 64
QN = 64
K = 8
BLKC = 16384
NB = 62
SEGW = 128
SEGB = BLKC // SEGW
NSEGP = NB * SEGB
NCAND = K * SEGW
VALID_LAST = POOL - (NB - 1) * BLKC
NEG = -1e30
BIG = 2**30


def _k1_body(q_ref, kt_ref, out_ref):
    b = pl.program_id(0)
    kt = kt_ref[...]
    scores = lax.dot_general(kt, q_ref[...], (((0,), (1,)), ((), ())),
                             preferred_element_type=jnp.float32)

    @pl.when(b < NB - 1)
    def _():
        out_ref[...] = jnp.max(scores.reshape(SEGB, SEGW, QN), axis=1)

    @pl.when(b == NB - 1)
    def _():
        c = lax.broadcasted_iota(jnp.int32, (BLKC, QN), 0)
        s = jnp.where(c >= VALID_LAST, NEG, scores)
        out_ref[...] = jnp.max(s.reshape(SEGB, SEGW, QN), axis=1)


def _k2_body(seg_ref, ids_ref):
    x = seg_ref[...]
    iota = lax.broadcasted_iota(jnp.int32, (NSEGP, QN), 0)
    ids = []
    for _ in range(K):
        m = jnp.max(x, axis=0, keepdims=True)
        sel = x == m
        idx = jnp.min(jnp.where(sel, iota, BIG), axis=0, keepdims=True)
        ids.append(idx)
        x = jnp.where(iota == idx, NEG, x)
    ids_ref[...] = jnp.concatenate(ids, axis=0)


def _k4_body(seg_ref, q_ref, k0, k1, k2, k3, k4, k5, k6, k7,
             idx_ref, w_ref):
    r = pl.program_id(0)
    qr = q_ref[0]
    wcat = jnp.concatenate([k[...] for k in (k0, k1, k2, k3, k4, k5, k6, k7)],
                           axis=1)
    sc = lax.dot_general(qr, wcat, (((1,), (0,)), ((), ())),
                         preferred_element_type=jnp.float32)
    j = lax.broadcasted_iota(jnp.int32, (1, SEGW), 1)
    ci = jnp.concatenate(
        [seg_ref[r, i] * SEGW + j for i in range(K)], axis=1)
    sc = jnp.where(ci < POOL, sc, NEG)
    vals, idxs = [], []
    for _ in range(K):
        m = jnp.max(sc, axis=1, keepdims=True)
        sel = sc == m
        pidx = jnp.min(jnp.where(sel, ci, BIG), axis=1, keepdims=True)
        vals.append(m)
        idxs.append(pidx)
        sc = jnp.where(ci == pidx, NEG, sc)
    v = jnp.concatenate(vals, axis=1)
    idx_ref[...] = jnp.concatenate(idxs, axis=1).reshape(1, 1, K)
    e = jnp.exp(v - v[:, 0:1])
    w_ref[...] = (e / jnp.sum(e, axis=1, keepdims=True)).reshape(1, 1, K)


def _k5_body(c_ref, u_ref, w_ref, W_ref, p0, p1, p2, p3, p4, p5, p6, p7,
             out_ref):
    r = pl.program_id(0)
    pcat = jnp.concatenate([p[...] for p in (p0, p1, p2, p3, p4, p5, p6, p7)],
                           axis=1)
    col = jnp.concatenate(
        [jnp.full((1, 1), i * SEGW, jnp.int32) + u_ref[r, i]
         for i in range(K)], axis=0)
    lane = lax.broadcasted_iota(jnp.int32, (K, K * SEGW), 1)
    onehot = (lane == col).astype(jnp.float32)
    stack = lax.dot_general(onehot, pcat, (((1,), (1,)), ((), ())),
                            preferred_element_type=jnp.float32)
    agg = jnp.sum(stack * w_ref[0], axis=0).reshape(1, D)
    out = lax.dot_general(agg, W_ref[...], (((1,), (1,)), ((), ())),
                          preferred_element_type=jnp.float32)
    out_ref[...] = out.reshape(1, 1, D)


@jax.jit
def kernel(query, pool, keys, W):
    B, S, _ = query.shape
    q = query.reshape(QN, D)
    kt = keys.T
    pt = pool.T

    seg_max = pl.pallas_call(
        _k1_body,
        grid=(NB,),
        in_specs=[
            pl.BlockSpec((QN, D), lambda b: (0, 0)),
            pl.BlockSpec((D, BLKC), lambda b: (0, b)),
        ],
        out_specs=pl.BlockSpec((SEGB, QN), lambda b: (b, 0)),
        out_shape=jax.ShapeDtypeStruct((NSEGP, QN), jnp.float32),
    )(q, kt)

    seg_ids_t = pl.pallas_call(
        _k2_body,
        out_shape=jax.ShapeDtypeStruct((K, QN), jnp.int32),
    )(seg_max)

    seg = seg_ids_t.T
    q3 = q.reshape(QN, 1, D)

    def _seg_map(i):
        return lambda r, seg_ref: (0, seg_ref[r, i])

    final_idx3, weights3 = pl.pallas_call(
        _k4_body,
        grid_spec=pltpu.PrefetchScalarGridSpec(
            num_scalar_prefetch=1,
            grid=(QN,),
            in_specs=[pl.BlockSpec((1, 1, D), lambda r, s: (r, 0, 0))] +
                     [pl.BlockSpec((D, SEGW), _seg_map(i)) for i in range(K)],
            out_specs=[pl.BlockSpec((1, 1, K), lambda r, s: (r, 0, 0)),
                       pl.BlockSpec((1, 1, K), lambda r, s: (r, 0, 0))],
        ),
        out_shape=[jax.ShapeDtypeStruct((QN, 1, K), jnp.int32),
                   jax.ShapeDtypeStruct((QN, 1, K), jnp.float32)],
    )(seg, q3, *([kt] * K))

    final_idx = final_idx3.reshape(QN, K)
    cblk = final_idx // SEGW
    uoff = final_idx % SEGW
    wb = jnp.broadcast_to(weights3.reshape(QN, K, 1), (QN, K, D))

    def _col_map(i):
        return lambda r, c_ref, u_ref: (0, c_ref[r, i])

    out3 = pl.pallas_call(
        _k5_body,
        grid_spec=pltpu.PrefetchScalarGridSpec(
            num_scalar_prefetch=2,
            grid=(QN,),
            in_specs=[pl.BlockSpec((1, K, D), lambda r, c, u: (r, 0, 0)),
                      pl.BlockSpec((D, D), lambda r, c, u: (0, 0))] +
                     [pl.BlockSpec((D, SEGW), _col_map(i)) for i in range(K)],
            out_specs=pl.BlockSpec((1, 1, D), lambda r, c, u: (r, 0, 0)),
        ),
        out_shape=jax.ShapeDtypeStruct((QN, 1, D), jnp.float32),
    )(cblk, uoff, wb, W, *([pt] * K))

    return out3.reshape(B, S, D)


# speedup vs baseline: 2.2649x; 1.0139x over previous
import functools

import jax
import jax.numpy as jnp
from jax import lax
from jax.experimental import pallas as pl
from jax.experimental.pallas import tpu as pltpu

POOL = 1000000
D = 64
QN = 64
K = 8
BLKC = 32768
NB = 31
SEGW = 128
SEGB = BLKC // SEGW
NSEGP = NB * SEGB
NCAND = K * SEGW
VALID_LAST = POOL - (NB - 1) * BLKC
NEG = -1e30
BIG = 2**30


def _k1_body(q_ref, kt_ref, out_ref):
    b = pl.program_id(0)
    kt = kt_ref[...]
    scores = lax.dot_general(kt, q_ref[...], (((0,), (1,)), ((), ())),
                             preferred_element_type=jnp.float32)

    @pl.when(b < NB - 1)
    def _():
        out_ref[...] = jnp.max(scores.reshape(SEGB, SEGW, QN), axis=1)

    @pl.when(b == NB - 1)
    def _():
        c = lax.broadcasted_iota(jnp.int32, (BLKC, QN), 0)
        s = jnp.where(c >= VALID_LAST, NEG, scores)
        out_ref[...] = jnp.max(s.reshape(SEGB, SEGW, QN), axis=1)


def _k2_body(seg_ref, ids_ref):
    x = seg_ref[...]
    iota = lax.broadcasted_iota(jnp.int32, (NSEGP, QN), 0)
    ids = []
    for _ in range(K):
        m = jnp.max(x, axis=0, keepdims=True)
        sel = x == m
        idx = jnp.min(jnp.where(sel, iota, BIG), axis=0, keepdims=True)
        ids.append(idx)
        x = jnp.where(iota == idx, NEG, x)
    ids_ref[...] = jnp.concatenate(ids, axis=0)


def _k4_body(seg_ref, q_ref, k0, k1, k2, k3, k4, k5, k6, k7,
             idx_ref, w_ref):
    r = pl.program_id(0)
    qr = q_ref[0]
    wcat = jnp.concatenate([k[...] for k in (k0, k1, k2, k3, k4, k5, k6, k7)],
                           axis=1)
    sc = lax.dot_general(qr, wcat, (((1,), (0,)), ((), ())),
                         preferred_element_type=jnp.float32)
    j = lax.broadcasted_iota(jnp.int32, (1, SEGW), 1)
    ci = jnp.concatenate(
        [seg_ref[r, i] * SEGW + j for i in range(K)], axis=1)
    sc = jnp.where(ci < POOL, sc, NEG)
    vals, idxs = [], []
    for _ in range(K):
        m = jnp.max(sc, axis=1, keepdims=True)
        sel = sc == m
        pidx = jnp.min(jnp.where(sel, ci, BIG), axis=1, keepdims=True)
        vals.append(m)
        idxs.append(pidx)
        sc = jnp.where(ci == pidx, NEG, sc)
    v = jnp.concatenate(vals, axis=1)
    idx_ref[...] = jnp.concatenate(idxs, axis=1).reshape(1, 1, K)
    e = jnp.exp(v - v[:, 0:1])
    w_ref[...] = (e / jnp.sum(e, axis=1, keepdims=True)).reshape(1, 1, K)


def _k5_body(c_ref, u_ref, w_ref, W_ref, p0, p1, p2, p3, p4, p5, p6, p7,
             out_ref):
    r = pl.program_id(0)
    pcat = jnp.concatenate([p[...] for p in (p0, p1, p2, p3, p4, p5, p6, p7)],
                           axis=1)
    col = jnp.concatenate(
        [jnp.full((1, 1), i * SEGW, jnp.int32) + u_ref[r, i]
         for i in range(K)], axis=0)
    lane = lax.broadcasted_iota(jnp.int32, (K, K * SEGW), 1)
    onehot = (lane == col).astype(jnp.float32)
    stack = lax.dot_general(onehot, pcat, (((1,), (1,)), ((), ())),
                            preferred_element_type=jnp.float32)
    agg = jnp.sum(stack * w_ref[0], axis=0).reshape(1, D)
    out = lax.dot_general(agg, W_ref[...], (((1,), (1,)), ((), ())),
                          preferred_element_type=jnp.float32)
    out_ref[...] = out.reshape(1, 1, D)


@jax.jit
def kernel(query, pool, keys, W):
    B, S, _ = query.shape
    q = query.reshape(QN, D)
    kt = keys.T
    pt = pool.T

    seg_max = pl.pallas_call(
        _k1_body,
        grid=(NB,),
        in_specs=[
            pl.BlockSpec((QN, D), lambda b: (0, 0)),
            pl.BlockSpec((D, BLKC), lambda b: (0, b)),
        ],
        out_specs=pl.BlockSpec((SEGB, QN), lambda b: (b, 0)),
        out_shape=jax.ShapeDtypeStruct((NSEGP, QN), jnp.float32),
    )(q, kt)

    seg_ids_t = pl.pallas_call(
        _k2_body,
        out_shape=jax.ShapeDtypeStruct((K, QN), jnp.int32),
    )(seg_max)

    seg = seg_ids_t.T
    q3 = q.reshape(QN, 1, D)

    def _seg_map(i):
        return lambda r, seg_ref: (0, seg_ref[r, i])

    final_idx3, weights3 = pl.pallas_call(
        _k4_body,
        grid_spec=pltpu.PrefetchScalarGridSpec(
            num_scalar_prefetch=1,
            grid=(QN,),
            in_specs=[pl.BlockSpec((1, 1, D), lambda r, s: (r, 0, 0))] +
                     [pl.BlockSpec((D, SEGW), _seg_map(i)) for i in range(K)],
            out_specs=[pl.BlockSpec((1, 1, K), lambda r, s: (r, 0, 0)),
                       pl.BlockSpec((1, 1, K), lambda r, s: (r, 0, 0))],
        ),
        out_shape=[jax.ShapeDtypeStruct((QN, 1, K), jnp.int32),
                   jax.ShapeDtypeStruct((QN, 1, K), jnp.float32)],
    )(seg, q3, *([kt] * K))

    final_idx = final_idx3.reshape(QN, K)
    cblk = final_idx // SEGW
    uoff = final_idx % SEGW
    wb = jnp.broadcast_to(weights3.reshape(QN, K, 1), (QN, K, D))

    def _col_map(i):
        return lambda r, c_ref, u_ref: (0, c_ref[r, i])

    out3 = pl.pallas_call(
        _k5_body,
        grid_spec=pltpu.PrefetchScalarGridSpec(
            num_scalar_prefetch=2,
            grid=(QN,),
            in_specs=[pl.BlockSpec((1, K, D), lambda r, c, u: (r, 0, 0)),
                      pl.BlockSpec((D, D), lambda r, c, u: (0, 0))] +
                     [pl.BlockSpec((D, SEGW), _col_map(i)) for i in range(K)],
            out_specs=pl.BlockSpec((1, 1, D), lambda r, c, u: (r, 0, 0)),
        ),
        out_shape=jax.ShapeDtypeStruct((QN, 1, D), jnp.float32),
    )(cblk, uoff, wb, W, *([pt] * K))

    return out3.reshape(B, S, D)
